# baseline (speedup 1.0000x reference)
import functools

import numpy as np

import jax
import jax.numpy as jnp
from jax import lax
from jax.experimental import pallas as pl
from jax.experimental.pallas import tpu as pltpu
from jax.experimental.pallas import tpu_sc as plsc

GRID = 128
NSEG = 16 * 16 * 16
FKEYS = GRID * GRID * GRID
BM_WORDS = FKEYS // 32
SH = BM_WORDS // 4

BR = 8192
NC, NS, L = 2, 16, 16

I32_MIN = np.int32(-(2**31))
I32_MAX = np.int32(2**31 - 1)


def _skey_of(p_bits):
    sk = jnp.where(p_bits < 0, p_bits ^ np.int32(0x7FFFFFFF), p_bits)
    return jnp.where(p_bits == I32_MIN, np.int32(0), sk)


def _bkey_of(fk):
    return ((fk >> 17) << 8) | (((fk >> 10) & 15) << 4) | ((fk >> 3) & 15)


def _t1d(x):
    xf = x if x.dtype == jnp.float32 else lax.bitcast_convert_type(x, jnp.float32)
    r = jnp.transpose(xf).reshape((BR,))
    return r if x.dtype == jnp.float32 else lax.bitcast_convert_type(r, x.dtype)


def _tc1_body(n_real, fea_ref, coord_ref, tcoord_ref, wup_ref, bup_ref,
              wcls_ref, bcls_ref, p2d_out, pv_out, fk_out, tk_out):
    i = pl.program_id(0)
    x = fea_ref[...]
    h = jnp.maximum(
        jnp.dot(x, wup_ref[...], preferred_element_type=jnp.float32)
        + bup_ref[...], 0.0)
    p = (jnp.dot(h, wcls_ref[...], preferred_element_type=jnp.float32)
         + bcls_ref[...])
    p2d_out[...] = p
    pv_out[...] = _t1d(p)

    rows = i * BR + lax.broadcasted_iota(jnp.int32, (BR, 1), 0)
    c = coord_ref[...]
    fk = ((c[:, 0:1] * GRID + (c[:, 1:2] >> 3)) * GRID + (c[:, 2:3] >> 3)) \
        * GRID + (c[:, 3:4] >> 3)
    fk_out[...] = _t1d(jnp.where(rows < n_real[0], fk, I32_MAX))

    t = tcoord_ref[...]
    tk = ((t[:, 0:1] * GRID + (t[:, 1:2] >> 3)) * GRID + (t[:, 2:3] >> 3)) \
        * GRID + (t[:, 3:4] >> 3)
    tk_out[...] = _t1d(jnp.where(rows < n_real[1], tk, I32_MAX))


def _sc_body(np_total, pv_h, fkey_h, tkey_h, seg_h, mem_h,
             vpf, vkey, vidx, vout, vstg, table, shared, bncw, bncv, dsem):
    cid = lax.axis_index("c")
    sid = lax.axis_index("s")
    ch = np_total // NS
    nv = ch // L
    base = sid * ch
    lane = lax.iota(jnp.int32, L)

    def fill_table(nwords, val):
        v = jnp.full((L,), val, jnp.int32)

        def z(i, _):
            for u in range(8):
                table[pl.ds((i * 8 + u) * L, L)] = v
            return 0
        lax.fori_loop(0, nwords // (8 * L), z, 0)

    def merge_dups(key, val, combine, identity):
        bncw[...] = key
        bncv[...] = val
        acc = val
        for s in range(1, L):
            pidx = (lane + s) & (L - 1)
            kp = plsc.load_gather(bncw, [pidx])
            vp = plsc.load_gather(bncv, [pidx])
            acc = combine(acc, jnp.where(kp == key, vp, identity))
        return acc

    def scatter_combine(idx, val, combine, identity):
        det = idx & (4096 - 1)
        plsc.store_scatter(vout, [det], lane)
        dup = jnp.any(plsc.load_gather(vout, [det]) != lane)
        val = lax.cond(dup,
                       lambda: merge_dups(idx, val, combine, identity),
                       lambda: val)
        cur = plsc.load_gather(table, [idx])
        plsc.store_scatter(table, [idx], combine(cur, val))

    @pl.when(cid == 0)
    def _seg_max():
        fill_table(NSEG, I32_MIN)
        pltpu.sync_copy(pv_h.at[pl.ds(base, ch)], vpf)
        pltpu.sync_copy(fkey_h.at[pl.ds(base, ch)], vidx)

        def scat(i, _):
            fk = vidx[pl.ds(i * L, L)]
            m = fk < FKEYS
            kc = jnp.where(m, _bkey_of(fk), 0)
            bits = plsc.bitcast(vpf[pl.ds(i * L, L)], jnp.int32)
            vm = jnp.where(m, _skey_of(bits), I32_MIN)
            scatter_combine(kc, vm, jnp.maximum, I32_MIN)
            return 0
        lax.fori_loop(0, nv, scat, 0)

        pltpu.sync_copy(table.at[pl.ds(0, NSEG)],
                        shared.at[pl.ds(sid * SH, NSEG)])
        plsc.subcore_barrier()
        sl = NSEG // NS
        off = sid * sl
        cps = [pltpu.async_copy(shared.at[pl.ds(j * SH + off, sl)],
                                vout.at[pl.ds(j * sl, sl)], dsem)
               for j in range(NS)]
        for c in cps:
            c.wait()

        def mg(i, _):
            acc = vout[pl.ds(i * L, L)]
            for j in range(1, NS):
                acc = jnp.maximum(acc, vout[pl.ds(j * sl + i * L, L)])
            vkey[pl.ds(i * L, L)] = acc
            return 0
        lax.fori_loop(0, sl // L, mg, 0)
        pltpu.sync_copy(vkey.at[pl.ds(0, sl)],
                        shared.at[pl.ds(NS * SH + off, sl)])
        plsc.subcore_barrier()
        pltpu.sync_copy(shared.at[pl.ds(NS * SH, NSEG)],
                        table.at[pl.ds(0, NSEG)])

        def gb(i, _):
            fk = vidx[pl.ds(i * L, L)]
            kc = jnp.where(fk < FKEYS, _bkey_of(fk), 0)
            vout[pl.ds(i * L, L)] = plsc.load_gather(table, [kc])
            return 0
        lax.fori_loop(0, nv, gb, 0)
        pltpu.sync_copy(vout, seg_h.at[pl.ds(base, ch)])

    @pl.when(cid == 1)
    def _member():
        fill_table(BM_WORDS, 0)
        pltpu.sync_copy(tkey_h.at[pl.ds(base, ch)], vkey)

        def scat(i, _):
            k = vkey[pl.ds(i * L, L)]
            m = k < FKEYS
            wi = jnp.where(m, k >> 5, 0)
            bit = jnp.where(m, jnp.int32(1) << (k & 31), 0)
            scatter_combine(wi, bit, jnp.bitwise_or, 0)
            return 0
        lax.fori_loop(0, nv, scat, 0)

        sl = SH // NS
        off = sid * sl
        for h in (0, 1, 2, 3):
            pltpu.sync_copy(table.at[pl.ds(h * SH, SH)],
                            shared.at[pl.ds(sid * SH, SH)])
            plsc.subcore_barrier()
            cps = [pltpu.async_copy(shared.at[pl.ds(j * SH + off, sl)],
                                    vstg.at[pl.ds(j * sl, sl)], dsem)
                   for j in range(NS)]
            for c in cps:
                c.wait()

            def og(i, _):
                acc = vstg[pl.ds(i * L, L)]
                for j in range(1, NS):
                    acc = acc | vstg[pl.ds(j * sl + i * L, L)]
                vout[pl.ds(i * L, L)] = acc
                return 0
            lax.fori_loop(0, sl // L, og, 0)
            pltpu.sync_copy(vout.at[pl.ds(0, sl)],
                            shared.at[pl.ds(NS * SH + off, sl)])
            plsc.subcore_barrier()
            pltpu.sync_copy(shared.at[pl.ds(NS * SH, SH)],
                            table.at[pl.ds(h * SH, SH)])
            plsc.subcore_barrier()

        pltpu.sync_copy(fkey_h.at[pl.ds(base, ch)], vidx)

        def probe(i, _):
            k = vidx[pl.ds(i * L, L)]
            m = k < FKEYS
            wi = jnp.where(m, k >> 5, 0)
            w = plsc.load_gather(table, [wi])
            bit = (w >> (k & 31)) & 1
            vout[pl.ds(i * L, L)] = jnp.where(m, bit, 0)
            return 0
        lax.fori_loop(0, nv, probe, 0)
        pltpu.sync_copy(vout, mem_h.at[pl.ds(base, ch)])


def _tc23_body(consts, pv_ref, seg_ref, mem_ref, k_ref,
               fea_ref, wup_ref, bup_ref, wcls_ref, bcls_ref, out_ref,
               thr_ref):
    n, npad = consts
    i = pl.program_id(0)

    @pl.when(i == 0)
    def _select():
        bits = lax.bitcast_convert_type(pv_ref[...], jnp.int32)
        sk = _skey_of(bits)
        ub = lax.bitcast_convert_type(sk, jnp.uint32) ^ jnp.uint32(0x80000000)
        rows = lax.broadcasted_iota(jnp.int32, (npad,), 0)
        mask = (sk != seg_ref[...]) & (rows < n)
        mk = jnp.where(mask, ub, jnp.uint32(0xFFFFFFFF))
        k = k_ref[0]

        def step(b, ans):
            bit = lax.shift_right_logical(jnp.uint32(0x80000000),
                                          b.astype(jnp.uint32))
            cand = ans | bit
            cnt = jnp.sum((mk < cand).astype(jnp.int32))
            return jnp.where(cnt < k, cand, ans)
        thr = lax.fori_loop(0, 32, step, jnp.uint32(0))
        thr_ref[0] = lax.bitcast_convert_type(thr, jnp.int32)

    blk = pl.ds(i * BR, BR)
    bits = lax.bitcast_convert_type(pv_ref[blk], jnp.int32)
    sk = _skey_of(bits)
    ub = lax.bitcast_convert_type(sk, jnp.uint32) ^ jnp.uint32(0x80000000)
    thr_u = lax.bitcast_convert_type(thr_ref[0], jnp.uint32)
    keep = (ub > thr_u) | (sk == seg_ref[blk]) | (mem_ref[blk] != 0)

    h = jnp.maximum(
        jnp.dot(fea_ref[...], wup_ref[...], preferred_element_type=jnp.float32)
        + bup_ref[...], 0.0)
    keepf = keep.astype(jnp.float32).reshape((BR, 1))
    out_ref[...] = jnp.where(keepf != 0.0, h, 0.0)


def kernel(fea_F, fea_C, target_C, target_points_num, W_up, b_up, W_cls, b_cls):
    n = fea_F.shape[0]
    nt = target_C.shape[0]
    c_in = fea_F.shape[1]
    c_out = W_up.shape[1]
    npad = ((n + BR - 1) // BR) * BR
    align = L * NS * 8
    if npad % align != 0:
        npad = ((npad + align - 1) // align) * align
    g = npad // BR

    tcoord_p = jnp.pad(target_C, ((0, npad - nt), (0, 0)))

    p2d, pv, fkv, tkv = pl.pallas_call(
        functools.partial(_tc1_body, (n, nt)),
        grid=(g,),
        in_specs=[
            pl.BlockSpec((BR, c_in), lambda i: (i, 0)),
            pl.BlockSpec((BR, 4), lambda i: (i, 0)),
            pl.BlockSpec((BR, 4), lambda i: (i, 0)),
            pl.BlockSpec((c_in, c_out), lambda i: (0, 0)),
            pl.BlockSpec((1, c_out), lambda i: (0, 0)),
            pl.BlockSpec((c_out, 1), lambda i: (0, 0)),
            pl.BlockSpec((1, 1), lambda i: (0, 0)),
        ],
        out_specs=[
            pl.BlockSpec((BR, 1), lambda i: (i, 0)),
            pl.BlockSpec((BR,), lambda i: (i,)),
            pl.BlockSpec((BR,), lambda i: (i,)),
            pl.BlockSpec((BR,), lambda i: (i,)),
        ],
        out_shape=[
            jax.ShapeDtypeStruct((n, 1), jnp.float32),
            jax.ShapeDtypeStruct((npad,), jnp.float32),
            jax.ShapeDtypeStruct((npad,), jnp.int32),
            jax.ShapeDtypeStruct((npad,), jnp.int32),
        ],
    )(fea_F, fea_C, tcoord_p, W_up, b_up.reshape(1, -1), W_cls,
      b_cls.reshape(1, 1))

    mesh = plsc.VectorSubcoreMesh(core_axis_name="c", subcore_axis_name="s",
                                  num_cores=NC, num_subcores=NS)
    ch = npad // NS
    seg, mem = pl.kernel(
        functools.partial(_sc_body, npad),
        out_type=[jax.ShapeDtypeStruct((npad,), jnp.int32),
                  jax.ShapeDtypeStruct((npad,), jnp.int32)],
        mesh=mesh,
        scratch_types=[
            pltpu.VMEM((ch,), jnp.float32),
            pltpu.VMEM((ch,), jnp.int32),
            pltpu.VMEM((ch,), jnp.int32),
            pltpu.VMEM((ch,), jnp.int32),
            pltpu.VMEM((SH,), jnp.int32),
            pltpu.VMEM((BM_WORDS,), jnp.int32),
            pltpu.VMEM_SHARED(((NS + 1) * SH,), jnp.int32),
            pltpu.VMEM((L,), jnp.int32),
            pltpu.VMEM((L,), jnp.int32),
            pltpu.SemaphoreType.DMA,
        ],
        compiler_params=pltpu.CompilerParams(needs_layout_passes=False),
    )(pv, fkv, tkv)


    k_arr = jnp.asarray(n - target_points_num, jnp.int32).reshape(1)
    pruned = pl.pallas_call(
        functools.partial(_tc23_body, (n, npad)),
        grid=(g,),
        in_specs=[
            pl.BlockSpec((npad,), lambda i: (0,)),
            pl.BlockSpec((npad,), lambda i: (0,)),
            pl.BlockSpec((npad,), lambda i: (0,)),
            pl.BlockSpec(memory_space=pltpu.SMEM),
            pl.BlockSpec((BR, c_in), lambda i: (i, 0)),
            pl.BlockSpec((c_in, c_out), lambda i: (0, 0)),
            pl.BlockSpec((1, c_out), lambda i: (0, 0)),
            pl.BlockSpec((c_out, 1), lambda i: (0, 0)),
            pl.BlockSpec((1, 1), lambda i: (0, 0)),
        ],
        out_specs=pl.BlockSpec((BR, c_out), lambda i: (i, 0)),
        out_shape=jax.ShapeDtypeStruct((n, c_out), jnp.float32),
        scratch_shapes=[pltpu.SMEM((1,), jnp.int32)],
    )(pv, seg, mem, k_arr,
      fea_F, W_up, b_up.reshape(1, -1), W_cls, b_cls.reshape(1, 1))

    return pruned, p2d, mem[:n].astype(bool)


# speedup vs baseline: 1.0061x; 1.0061x over previous
import functools

import numpy as np

import jax
import jax.numpy as jnp
from jax import lax
from jax.experimental import pallas as pl
from jax.experimental.pallas import tpu as pltpu
from jax.experimental.pallas import tpu_sc as plsc

GRID = 128
NSEG = 16 * 16 * 16
FKEYS = GRID * GRID * GRID
BM_WORDS = FKEYS // 32
SH = BM_WORDS // 4

BR = 4096
NC, NS, L = 2, 16, 16

I32_MIN = np.int32(-(2**31))
I32_MAX = np.int32(2**31 - 1)


def _skey_of(p_bits):
    sk = jnp.where(p_bits < 0, p_bits ^ np.int32(0x7FFFFFFF), p_bits)
    return jnp.where(p_bits == I32_MIN, np.int32(0), sk)


def _bkey_of(fk):
    return ((fk >> 17) << 8) | (((fk >> 10) & 15) << 4) | ((fk >> 3) & 15)


def _t1d(x):
    xf = x if x.dtype == jnp.float32 else lax.bitcast_convert_type(x, jnp.float32)
    r = jnp.transpose(xf).reshape((BR,))
    return r if x.dtype == jnp.float32 else lax.bitcast_convert_type(r, x.dtype)


def _tc1_body(n_real, fea_ref, coord_ref, tcoord_ref, wup_ref, bup_ref,
              wcls_ref, bcls_ref, p2d_out, pv_out, fk_out, tk_out):
    i = pl.program_id(0)
    x = fea_ref[...]
    h = jnp.maximum(
        jnp.dot(x, wup_ref[...], preferred_element_type=jnp.float32)
        + bup_ref[...], 0.0)
    p = (jnp.dot(h, wcls_ref[...], preferred_element_type=jnp.float32)
         + bcls_ref[...])
    p2d_out[...] = p
    pv_out[...] = _t1d(p)

    rows = i * BR + lax.broadcasted_iota(jnp.int32, (BR, 1), 0)
    c = coord_ref[...]
    fk = ((c[:, 0:1] * GRID + (c[:, 1:2] >> 3)) * GRID + (c[:, 2:3] >> 3)) \
        * GRID + (c[:, 3:4] >> 3)
    fk_out[...] = _t1d(jnp.where(rows < n_real[0], fk, I32_MAX))

    t = tcoord_ref[...]
    tk = ((t[:, 0:1] * GRID + (t[:, 1:2] >> 3)) * GRID + (t[:, 2:3] >> 3)) \
        * GRID + (t[:, 3:4] >> 3)
    tk_out[...] = _t1d(jnp.where(rows < n_real[1], tk, I32_MAX))


def _sc_body(np_total, pv_h, fkey_h, tkey_h, seg_h, mem_h,
             vpf, vkey, vidx, vout, vstg, table, shared, bncw, bncv, dsem):
    cid = lax.axis_index("c")
    sid = lax.axis_index("s")
    ch = np_total // NS
    nv = ch // L
    base = sid * ch
    lane = lax.iota(jnp.int32, L)

    def fill_table(nwords, val):
        v = jnp.full((L,), val, jnp.int32)

        def z(i, _):
            for u in range(8):
                table[pl.ds((i * 8 + u) * L, L)] = v
            return 0
        lax.fori_loop(0, nwords // (8 * L), z, 0)

    def merge_dups(key, val, combine, identity):
        bncw[...] = key
        bncv[...] = val
        acc = val
        for s in range(1, L):
            pidx = (lane + s) & (L - 1)
            kp = plsc.load_gather(bncw, [pidx])
            vp = plsc.load_gather(bncv, [pidx])
            acc = combine(acc, jnp.where(kp == key, vp, identity))
        return acc

    def scatter_combine(idx, val, combine, identity):
        det = idx & (4096 - 1)
        plsc.store_scatter(vout, [det], lane)
        dup = jnp.any(plsc.load_gather(vout, [det]) != lane)
        val = lax.cond(dup,
                       lambda: merge_dups(idx, val, combine, identity),
                       lambda: val)
        cur = plsc.load_gather(table, [idx])
        plsc.store_scatter(table, [idx], combine(cur, val))

    @pl.when(cid == 0)
    def _seg_max():
        fill_table(NSEG, I32_MIN)
        pltpu.sync_copy(pv_h.at[pl.ds(base, ch)], vpf)
        pltpu.sync_copy(fkey_h.at[pl.ds(base, ch)], vidx)

        def scat(i, _):
            fk = vidx[pl.ds(i * L, L)]
            m = fk < FKEYS
            kc = jnp.where(m, _bkey_of(fk), 0)
            bits = plsc.bitcast(vpf[pl.ds(i * L, L)], jnp.int32)
            vm = jnp.where(m, _skey_of(bits), I32_MIN)
            scatter_combine(kc, vm, jnp.maximum, I32_MIN)
            return 0
        lax.fori_loop(0, nv, scat, 0)

        pltpu.sync_copy(table.at[pl.ds(0, NSEG)],
                        shared.at[pl.ds(sid * SH, NSEG)])
        plsc.subcore_barrier()
        sl = NSEG // NS
        off = sid * sl
        cps = [pltpu.async_copy(shared.at[pl.ds(j * SH + off, sl)],
                                vout.at[pl.ds(j * sl, sl)], dsem)
               for j in range(NS)]
        for c in cps:
            c.wait()

        def mg(i, _):
            acc = vout[pl.ds(i * L, L)]
            for j in range(1, NS):
                acc = jnp.maximum(acc, vout[pl.ds(j * sl + i * L, L)])
            vkey[pl.ds(i * L, L)] = acc
            return 0
        lax.fori_loop(0, sl // L, mg, 0)
        pltpu.sync_copy(vkey.at[pl.ds(0, sl)],
                        shared.at[pl.ds(NS * SH + off, sl)])
        plsc.subcore_barrier()
        pltpu.sync_copy(shared.at[pl.ds(NS * SH, NSEG)],
                        table.at[pl.ds(0, NSEG)])

        def gb(i, _):
            fk = vidx[pl.ds(i * L, L)]
            kc = jnp.where(fk < FKEYS, _bkey_of(fk), 0)
            vout[pl.ds(i * L, L)] = plsc.load_gather(table, [kc])
            return 0
        lax.fori_loop(0, nv, gb, 0)
        pltpu.sync_copy(vout, seg_h.at[pl.ds(base, ch)])

    @pl.when(cid == 1)
    def _member():
        fill_table(BM_WORDS, 0)
        pltpu.sync_copy(tkey_h.at[pl.ds(base, ch)], vkey)

        def scat(i, _):
            k = vkey[pl.ds(i * L, L)]
            m = k < FKEYS
            wi = jnp.where(m, k >> 5, 0)
            bit = jnp.where(m, jnp.int32(1) << (k & 31), 0)
            scatter_combine(wi, bit, jnp.bitwise_or, 0)
            return 0
        lax.fori_loop(0, nv, scat, 0)

        sl = SH // NS
        off = sid * sl
        for h in (0, 1, 2, 3):
            pltpu.sync_copy(table.at[pl.ds(h * SH, SH)],
                            shared.at[pl.ds(sid * SH, SH)])
            plsc.subcore_barrier()
            cps = [pltpu.async_copy(shared.at[pl.ds(j * SH + off, sl)],
                                    vstg.at[pl.ds(j * sl, sl)], dsem)
                   for j in range(NS)]
            for c in cps:
                c.wait()

            def og(i, _):
                acc = vstg[pl.ds(i * L, L)]
                for j in range(1, NS):
                    acc = acc | vstg[pl.ds(j * sl + i * L, L)]
                vout[pl.ds(i * L, L)] = acc
                return 0
            lax.fori_loop(0, sl // L, og, 0)
            pltpu.sync_copy(vout.at[pl.ds(0, sl)],
                            shared.at[pl.ds(NS * SH + off, sl)])
            plsc.subcore_barrier()
            pltpu.sync_copy(shared.at[pl.ds(NS * SH, SH)],
                            table.at[pl.ds(h * SH, SH)])
            plsc.subcore_barrier()

        pltpu.sync_copy(fkey_h.at[pl.ds(base, ch)], vidx)

        def probe(i, _):
            k = vidx[pl.ds(i * L, L)]
            m = k < FKEYS
            wi = jnp.where(m, k >> 5, 0)
            w = plsc.load_gather(table, [wi])
            bit = (w >> (k & 31)) & 1
            vout[pl.ds(i * L, L)] = jnp.where(m, bit, 0)
            return 0
        lax.fori_loop(0, nv, probe, 0)
        pltpu.sync_copy(vout, mem_h.at[pl.ds(base, ch)])


def _tc23_body(consts, pv_ref, seg_ref, mem_ref, k_ref,
               fea_ref, wup_ref, bup_ref, wcls_ref, bcls_ref, out_ref,
               thr_ref):
    n, npad = consts
    i = pl.program_id(0)

    @pl.when(i == 0)
    def _select():
        bits = lax.bitcast_convert_type(pv_ref[...], jnp.int32)
        sk = _skey_of(bits)
        ub = lax.bitcast_convert_type(sk, jnp.uint32) ^ jnp.uint32(0x80000000)
        rows = lax.broadcasted_iota(jnp.int32, (npad,), 0)
        mask = (sk != seg_ref[...]) & (rows < n)
        mk = jnp.where(mask, ub, jnp.uint32(0xFFFFFFFF))
        k = k_ref[0]

        def step(b, ans):
            bit = lax.shift_right_logical(jnp.uint32(0x80000000),
                                          b.astype(jnp.uint32))
            cand = ans | bit
            cnt = jnp.sum((mk < cand).astype(jnp.int32))
            return jnp.where(cnt < k, cand, ans)
        thr = lax.fori_loop(0, 32, step, jnp.uint32(0))
        thr_ref[0] = lax.bitcast_convert_type(thr, jnp.int32)

    blk = pl.ds(i * BR, BR)
    bits = lax.bitcast_convert_type(pv_ref[blk], jnp.int32)
    sk = _skey_of(bits)
    ub = lax.bitcast_convert_type(sk, jnp.uint32) ^ jnp.uint32(0x80000000)
    thr_u = lax.bitcast_convert_type(thr_ref[0], jnp.uint32)
    keep = (ub > thr_u) | (sk == seg_ref[blk]) | (mem_ref[blk] != 0)

    h = jnp.maximum(
        jnp.dot(fea_ref[...], wup_ref[...], preferred_element_type=jnp.float32)
        + bup_ref[...], 0.0)
    keepf = keep.astype(jnp.float32).reshape((BR, 1))
    out_ref[...] = jnp.where(keepf != 0.0, h, 0.0)


def kernel(fea_F, fea_C, target_C, target_points_num, W_up, b_up, W_cls, b_cls):
    n = fea_F.shape[0]
    nt = target_C.shape[0]
    c_in = fea_F.shape[1]
    c_out = W_up.shape[1]
    npad = ((n + BR - 1) // BR) * BR
    align = L * NS * 8
    if npad % align != 0:
        npad = ((npad + align - 1) // align) * align
    g = npad // BR

    tcoord_p = jnp.pad(target_C, ((0, npad - nt), (0, 0)))

    p2d, pv, fkv, tkv = pl.pallas_call(
        functools.partial(_tc1_body, (n, nt)),
        grid=(g,),
        in_specs=[
            pl.BlockSpec((BR, c_in), lambda i: (i, 0)),
            pl.BlockSpec((BR, 4), lambda i: (i, 0)),
            pl.BlockSpec((BR, 4), lambda i: (i, 0)),
            pl.BlockSpec((c_in, c_out), lambda i: (0, 0)),
            pl.BlockSpec((1, c_out), lambda i: (0, 0)),
            pl.BlockSpec((c_out, 1), lambda i: (0, 0)),
            pl.BlockSpec((1, 1), lambda i: (0, 0)),
        ],
        out_specs=[
            pl.BlockSpec((BR, 1), lambda i: (i, 0)),
            pl.BlockSpec((BR,), lambda i: (i,)),
            pl.BlockSpec((BR,), lambda i: (i,)),
            pl.BlockSpec((BR,), lambda i: (i,)),
        ],
        out_shape=[
            jax.ShapeDtypeStruct((n, 1), jnp.float32),
            jax.ShapeDtypeStruct((npad,), jnp.float32),
            jax.ShapeDtypeStruct((npad,), jnp.int32),
            jax.ShapeDtypeStruct((npad,), jnp.int32),
        ],
    )(fea_F, fea_C, tcoord_p, W_up, b_up.reshape(1, -1), W_cls,
      b_cls.reshape(1, 1))

    mesh = plsc.VectorSubcoreMesh(core_axis_name="c", subcore_axis_name="s",
                                  num_cores=NC, num_subcores=NS)
    ch = npad // NS
    seg, mem = pl.kernel(
        functools.partial(_sc_body, npad),
        out_type=[jax.ShapeDtypeStruct((npad,), jnp.int32),
                  jax.ShapeDtypeStruct((npad,), jnp.int32)],
        mesh=mesh,
        scratch_types=[
            pltpu.VMEM((ch,), jnp.float32),
            pltpu.VMEM((ch,), jnp.int32),
            pltpu.VMEM((ch,), jnp.int32),
            pltpu.VMEM((ch,), jnp.int32),
            pltpu.VMEM((SH,), jnp.int32),
            pltpu.VMEM((BM_WORDS,), jnp.int32),
            pltpu.VMEM_SHARED(((NS + 1) * SH,), jnp.int32),
            pltpu.VMEM((L,), jnp.int32),
            pltpu.VMEM((L,), jnp.int32),
            pltpu.SemaphoreType.DMA,
        ],
        compiler_params=pltpu.CompilerParams(needs_layout_passes=False),
    )(pv, fkv, tkv)


    k_arr = jnp.asarray(n - target_points_num, jnp.int32).reshape(1)
    pruned = pl.pallas_call(
        functools.partial(_tc23_body, (n, npad)),
        grid=(g,),
        in_specs=[
            pl.BlockSpec((npad,), lambda i: (0,)),
            pl.BlockSpec((npad,), lambda i: (0,)),
            pl.BlockSpec((npad,), lambda i: (0,)),
            pl.BlockSpec(memory_space=pltpu.SMEM),
            pl.BlockSpec((BR, c_in), lambda i: (i, 0)),
            pl.BlockSpec((c_in, c_out), lambda i: (0, 0)),
            pl.BlockSpec((1, c_out), lambda i: (0, 0)),
            pl.BlockSpec((c_out, 1), lambda i: (0, 0)),
            pl.BlockSpec((1, 1), lambda i: (0, 0)),
        ],
        out_specs=pl.BlockSpec((BR, c_out), lambda i: (i, 0)),
        out_shape=jax.ShapeDtypeStruct((n, c_out), jnp.float32),
        scratch_shapes=[pltpu.SMEM((1,), jnp.int32)],
    )(pv, seg, mem, k_arr,
      fea_F, W_up, b_up.reshape(1, -1), W_cls, b_cls.reshape(1, 1))

    return pruned, p2d, mem[:n].astype(bool)
